# baseline (speedup 1.0000x reference)
import functools

import jax
import jax.numpy as jnp
from jax import lax
from jax.experimental import pallas as pl
from jax.experimental.pallas import tpu as pltpu
from jax.experimental.pallas import tpu_sc as plsc

NC = 2
NS = 16
L = 16
NW = NC * NS

B, T, D = 4, 2048, 1024
N = B * T
RPW = N // NW
C = 32
NCH = RPW // C
KD = D // L


def _emb_body(x_hbm, tok_hbm, pos_hbm, out_hbm, idx_v, tok_v, pos_v, sem):
    c = lax.axis_index("c")
    s = lax.axis_index("s")
    wid = s * NC + c
    base = wid * RPW
    pos_base = (wid % (T // RPW)) * RPW

    pltpu.sync_copy(x_hbm.at[pl.ds(base, RPW)], idx_v)

    for j in range(NCH):
        gather = pltpu.async_copy(
            tok_hbm.at[idx_v.at[pl.ds(j * C, C)]], tok_v, sem)
        pltpu.sync_copy(pos_hbm.at[pl.ds(pos_base + j * C, C)], pos_v)
        gather.wait()

        def add_one(i, _):
            r = i // KD
            k = (i % KD) * L
            tok_v[r, pl.ds(k, L)] = tok_v[r, pl.ds(k, L)] + pos_v[r, pl.ds(k, L)]
            return 0

        lax.fori_loop(0, C * KD, add_one, 0)
        pltpu.sync_copy(tok_v, out_hbm.at[pl.ds(base + j * C, C)])


@jax.jit
def _emb(x_flat, tok_table, pos_table):
    mesh = plsc.VectorSubcoreMesh(
        core_axis_name="c", subcore_axis_name="s",
        num_cores=NC, num_subcores=NS)
    return pl.kernel(
        _emb_body,
        out_type=jax.ShapeDtypeStruct((N, D), jnp.float32),
        mesh=mesh,
        scratch_types=[
            pltpu.VMEM((RPW,), jnp.int32),
            pltpu.VMEM((C, D), jnp.float32),
            pltpu.VMEM((C, D), jnp.float32),
            pltpu.SemaphoreType.DMA,
        ],
    )(x_flat, tok_table, pos_table)


def kernel(x, tok_table, pos_table):
    b, t = x.shape
    x_flat = x.reshape(b * t).astype(jnp.int32)
    out = _emb(x_flat, tok_table, pos_table)
    return out.reshape(b, t, D)


# speedup vs baseline: 2.3246x; 2.3246x over previous
import functools

import jax
import jax.numpy as jnp
from jax import lax
from jax.experimental import pallas as pl
from jax.experimental.pallas import tpu as pltpu
from jax.experimental.pallas import tpu_sc as plsc

NC = 2
NS = 16
L = 16
NW = NC * NS

B, T, D = 4, 2048, 1024
N = B * T
RPW = N // NW
SPW = T // NW
C = 16
NCH = RPW // C
HPS = SPW // C
KD = D // L


def _emb_body(x_hbm, tok_hbm, pos_hbm, out_hbm,
              idx_v, pos_v, tok0_v, tok1_v, sg0, sg1, so0, so1):
    c = lax.axis_index("c")
    s = lax.axis_index("s")
    wid = s * NC + c
    tok_v = (tok0_v, tok1_v)
    sg = (sg0, sg1)
    so = (so0, so1)

    pltpu.sync_copy(pos_hbm.at[pl.ds(wid * SPW, SPW)], pos_v)
    for b in range(B):
        pltpu.sync_copy(x_hbm.at[pl.ds(b * T + wid * SPW, SPW)],
                        idx_v.at[pl.ds(b * SPW, SPW)])

    def gather(t, p):
        return pltpu.async_copy(
            tok_hbm.at[idx_v.at[pl.ds(t * C, C)]], tok_v[p], sg[p])

    def out_base(t):
        b, h = t // HPS, t % HPS
        return b * T + wid * SPW + h * C

    g = [None, None]
    o = [None, None]
    g[0] = gather(0, 0)
    for t in range(NCH):
        p = t % 2
        q = (t + 1) % 2
        if t + 1 < NCH:
            if o[q] is not None:
                o[q].wait()
                o[q] = None
            g[q] = gather(t + 1, q)
        g[p].wait()

        h = t % HPS
        buf = tok_v[p]

        @plsc.parallel_loop(0, C * KD, 1, unroll=8)
        def add_body(i):
            r = i // KD
            k = (i % KD) * L
            plsc.addupdate(buf.at[r, pl.ds(k, L)],
                           pos_v[h * C + r, pl.ds(k, L)])

        o[p] = pltpu.async_copy(buf, out_hbm.at[pl.ds(out_base(t), C)], so[p])
    o[0].wait()
    o[1].wait()


@jax.jit
def _emb(x_flat, tok_table, pos_table):
    mesh = plsc.VectorSubcoreMesh(
        core_axis_name="c", subcore_axis_name="s",
        num_cores=NC, num_subcores=NS)
    return pl.kernel(
        _emb_body,
        out_type=jax.ShapeDtypeStruct((N, D), jnp.float32),
        mesh=mesh,
        scratch_types=[
            pltpu.VMEM((RPW,), jnp.int32),
            pltpu.VMEM((SPW, D), jnp.float32),
            pltpu.VMEM((C, D), jnp.float32),
            pltpu.VMEM((C, D), jnp.float32),
            pltpu.SemaphoreType.DMA,
            pltpu.SemaphoreType.DMA,
            pltpu.SemaphoreType.DMA,
            pltpu.SemaphoreType.DMA,
        ],
    )(x_flat, tok_table, pos_table)


def kernel(x, tok_table, pos_table):
    b, t = x.shape
    x_flat = x.reshape(b * t).astype(jnp.int32)
    out = _emb(x_flat, tok_table, pos_table)
    return out.reshape(b, t, D)


# speedup vs baseline: 2.4376x; 1.0486x over previous
import functools

import jax
import jax.numpy as jnp
from jax import lax
from jax.experimental import pallas as pl
from jax.experimental.pallas import tpu as pltpu
from jax.experimental.pallas import tpu_sc as plsc

NC = 2
NS = 16
L = 16
NW = NC * NS

B, T, D = 4, 2048, 1024
N = B * T
RPW = N // NW
SPW = T // NW
C = 16
NCH = RPW // C
HPS = SPW // C
KD = D // L


DEPTH = 3


def _emb_body(x_hbm, tok_hbm, pos_hbm, out_hbm,
              idx_v, pos_v, tok0_v, tok1_v, tok2_v,
              sp, si, sg0, sg1, sg2, so0, so1, so2):
    c = lax.axis_index("c")
    s = lax.axis_index("s")
    wid = s * NC + c
    tok_v = (tok0_v, tok1_v, tok2_v)
    sg = (sg0, sg1, sg2)
    so = (so0, so1, so2)

    pos_cp = pltpu.async_copy(pos_hbm.at[pl.ds(wid * SPW, SPW)], pos_v, sp)
    idx_cps = [
        pltpu.async_copy(x_hbm.at[pl.ds(b * T + wid * SPW, SPW)],
                         idx_v.at[pl.ds(b * SPW, SPW)], si)
        for b in range(B)
    ]
    for cp in idx_cps:
        cp.wait()

    def gather(t, p):
        return pltpu.async_copy(
            tok_hbm.at[idx_v.at[pl.ds(t * C, C)]], tok_v[p], sg[p])

    def out_base(t):
        b, h = t // HPS, t % HPS
        return b * T + wid * SPW + h * C

    g = [None] * DEPTH
    o = [None] * DEPTH
    for t0 in range(DEPTH - 1):
        g[t0] = gather(t0, t0)
    for t in range(NCH):
        p = t % DEPTH
        nt = t + DEPTH - 1
        if nt < NCH:
            q = nt % DEPTH
            if o[q] is not None:
                o[q].wait()
                o[q] = None
            g[q] = gather(nt, q)
        g[p].wait()
        if t == 0:
            pos_cp.wait()

        h = t % HPS
        buf = tok_v[p]

        @plsc.parallel_loop(0, C * KD, 1, unroll=8)
        def add_body(i):
            r = i // KD
            k = (i % KD) * L
            plsc.addupdate(buf.at[r, pl.ds(k, L)],
                           pos_v[h * C + r, pl.ds(k, L)])

        o[p] = pltpu.async_copy(buf, out_hbm.at[pl.ds(out_base(t), C)], so[p])
    for cp in o:
        if cp is not None:
            cp.wait()


@jax.jit
def _emb(x_flat, tok_table, pos_table):
    mesh = plsc.VectorSubcoreMesh(
        core_axis_name="c", subcore_axis_name="s",
        num_cores=NC, num_subcores=NS)
    return pl.kernel(
        _emb_body,
        out_type=jax.ShapeDtypeStruct((N, D), jnp.float32),
        mesh=mesh,
        scratch_types=[
            pltpu.VMEM((RPW,), jnp.int32),
            pltpu.VMEM((SPW, D), jnp.float32),
            pltpu.VMEM((C, D), jnp.float32),
            pltpu.VMEM((C, D), jnp.float32),
            pltpu.VMEM((C, D), jnp.float32),
        ] + [pltpu.SemaphoreType.DMA] * (2 + 2 * DEPTH),
    )(x_flat, tok_table, pos_table)


def kernel(x, tok_table, pos_table):
    b, t = x.shape
    x_flat = x.reshape(b * t).astype(jnp.int32)
    out = _emb(x_flat, tok_table, pos_table)
    return out.reshape(b, t, D)


# speedup vs baseline: 2.4502x; 1.0052x over previous
import functools

import jax
import jax.numpy as jnp
from jax import lax
from jax.experimental import pallas as pl
from jax.experimental.pallas import tpu as pltpu
from jax.experimental.pallas import tpu_sc as plsc

NC = 2
NS = 16
L = 16
NW = NC * NS

B, T, D = 4, 2048, 1024
N = B * T
RPW = N // NW
SPW = T // NW
C = 16
NCH = RPW // C
HPS = SPW // C
KD = D // L


DEPTH = 3


def _emb_body(x_hbm, tok_hbm, pos_hbm, out_hbm,
              idx_v, pos_v, tok0_v, tok1_v, tok2_v,
              sp, si, sg0, sg1, sg2, so0, so1, so2):
    c = lax.axis_index("c")
    s = lax.axis_index("s")
    wid = s * NC + c
    tok_v = (tok0_v, tok1_v, tok2_v)
    sg = (sg0, sg1, sg2)
    so = (so0, so1, so2)

    pos_cp = pltpu.async_copy(pos_hbm.at[pl.ds(wid * SPW, SPW)], pos_v, sp)
    idx_cps = [
        pltpu.async_copy(x_hbm.at[b, pl.ds(wid * SPW, SPW)],
                         idx_v.at[pl.ds(b * SPW, SPW)], si)
        for b in range(B)
    ]
    for cp in idx_cps:
        cp.wait()

    def gather(t, p):
        return pltpu.async_copy(
            tok_hbm.at[idx_v.at[pl.ds(t * C, C)]], tok_v[p], sg[p])

    g = [None] * DEPTH
    o = [None] * DEPTH
    for t0 in range(DEPTH - 1):
        g[t0] = gather(t0, t0)
    for t in range(NCH):
        p = t % DEPTH
        nt = t + DEPTH - 1
        if nt < NCH:
            q = nt % DEPTH
            if o[q] is not None:
                o[q].wait()
                o[q] = None
            g[q] = gather(nt, q)
        g[p].wait()
        if t == 0:
            pos_cp.wait()

        b, h = t // HPS, t % HPS
        buf = tok_v[p]

        @plsc.parallel_loop(0, C * KD, 1, unroll=8)
        def add_body(i):
            r = i // KD
            k = (i % KD) * L
            plsc.addupdate(buf.at[r, pl.ds(k, L)],
                           pos_v[h * C + r, pl.ds(k, L)])

        o[p] = pltpu.async_copy(
            buf, out_hbm.at[b, pl.ds(wid * SPW + h * C, C)], so[p])
    for cp in o:
        if cp is not None:
            cp.wait()


@jax.jit
def _emb(x, tok_table, pos_table):
    mesh = plsc.VectorSubcoreMesh(
        core_axis_name="c", subcore_axis_name="s",
        num_cores=NC, num_subcores=NS)
    return pl.kernel(
        _emb_body,
        out_type=jax.ShapeDtypeStruct((B, T, D), jnp.float32),
        mesh=mesh,
        scratch_types=[
            pltpu.VMEM((RPW,), jnp.int32),
            pltpu.VMEM((SPW, D), jnp.float32),
            pltpu.VMEM((C, D), jnp.float32),
            pltpu.VMEM((C, D), jnp.float32),
            pltpu.VMEM((C, D), jnp.float32),
        ] + [pltpu.SemaphoreType.DMA] * (2 + 2 * DEPTH),
    )(x, tok_table, pos_table)


def kernel(x, tok_table, pos_table):
    return _emb(x.astype(jnp.int32), tok_table, pos_table)
